# baseline (speedup 1.0000x reference)
import functools

import jax
import jax.numpy as jnp
from jax import lax
from jax.experimental import pallas as pl
from jax.experimental.pallas import tpu as pltpu
from jax.experimental.pallas import tpu_sc as plsc

N = 10000
E = 320000
D_IN = 128
HID = 64
D_OUT = 2

NC = 2
NS = 16
NW = NC * NS
CHUNK = 128
EPW = 10368
NCH = EPW // CHUNK
N_PAD = 10112
RPT = N_PAD // NS
DEG_W = 8
DP = 8
BR = 1000


def _sc_mesh():
  return plsc.VectorSubcoreMesh(core_axis_name="c", subcore_axis_name="s")


@functools.partial(
    pl.kernel,
    out_type=jax.ShapeDtypeStruct((NC, N_PAD, DEG_W), jnp.float32),
    mesh=_sc_mesh(),
    compiler_params=pltpu.CompilerParams(use_tc_tiling_on_sc=False),
    scratch_types=[
        pltpu.VMEM((NCH, CHUNK), jnp.int32),
        pltpu.VMEM((CHUNK, DEG_W), jnp.float32),
        pltpu.VMEM_SHARED((N_PAD, DEG_W), jnp.float32),
    ],
)
def _deg_kernel(dst_hbm, ones_hbm, zeros_hbm, out_hbm, didx_v, ones_v, acc_sh):
  c = lax.axis_index("c")
  s = lax.axis_index("s")
  wid = s * NC + c
  pltpu.sync_copy(dst_hbm.at[pl.ds(wid * NCH, NCH)], didx_v)
  pltpu.sync_copy(ones_hbm, ones_v)
  pltpu.sync_copy(zeros_hbm.at[pl.ds(s * RPT, RPT)],
                  acc_sh.at[pl.ds(s * RPT, RPT)])
  plsc.subcore_barrier()

  @pl.loop(0, NCH)
  def _(j):
    pltpu.sync_copy(ones_v, acc_sh.at[didx_v.at[j]], add=True)

  plsc.subcore_barrier()
  pltpu.sync_copy(acc_sh.at[pl.ds(s * RPT, RPT)],
                  out_hbm.at[c, pl.ds(s * RPT, RPT)])


def _make_propagate(D):
  @functools.partial(
      pl.kernel,
      out_type=jax.ShapeDtypeStruct((NC, N_PAD, D), jnp.float32),
      mesh=_sc_mesh(),
      compiler_params=pltpu.CompilerParams(use_tc_tiling_on_sc=False),
      scratch_types=[
          pltpu.VMEM((NCH + 2, CHUNK), jnp.int32),
          pltpu.VMEM((NCH, CHUNK), jnp.int32),
          pltpu.VMEM((CHUNK, D), jnp.float32),
          pltpu.VMEM((CHUNK, D), jnp.float32),
          pltpu.VMEM((CHUNK, D), jnp.float32),
          pltpu.VMEM_SHARED((N, D), jnp.float32),
          pltpu.VMEM_SHARED((N_PAD, D), jnp.float32),
          pltpu.SemaphoreType.DMA,
          pltpu.SemaphoreType.DMA,
          pltpu.SemaphoreType.DMA,
      ],
  )
  def _propagate(table_hbm, src_hbm, dst_hbm, zeros_hbm, out_hbm,
                 sidx_v, didx_v, rows_0, rows_1, rows_2,
                 table_sh, acc_sh, sem_0, sem_1, sem_2):
    bufs = (rows_0, rows_1, rows_2)
    sems = (sem_0, sem_1, sem_2)
    c = lax.axis_index("c")
    s = lax.axis_index("s")
    wid = s * NC + c
    pltpu.sync_copy(src_hbm.at[pl.ds(wid * NCH, NCH)],
                    sidx_v.at[pl.ds(0, NCH)])
    pltpu.sync_copy(dst_hbm.at[pl.ds(wid * NCH, NCH)], didx_v)
    for t in range(2):
      for k in range(CHUNK // 16):
        sidx_v[NCH + t, pl.ds(k * 16, 16)] = jnp.zeros((16,), jnp.int32)
    pltpu.sync_copy(table_hbm.at[pl.ds(s * (N // NS), N // NS)],
                    table_sh.at[pl.ds(s * (N // NS), N // NS)])
    pltpu.sync_copy(zeros_hbm.at[pl.ds(s * RPT, RPT)],
                    acc_sh.at[pl.ds(s * RPT, RPT)])
    plsc.subcore_barrier()

    for b in range(2):
      pltpu.async_copy(table_sh.at[sidx_v.at[b]], bufs[b], sems[b])

    @pl.loop(0, NCH, step=3)
    def _(j):
      for b in range(3):
        nb = (b + 2) % 3
        pltpu.async_copy(table_sh.at[sidx_v.at[j + b + 2]], bufs[nb],
                         sems[nb])
        pltpu.make_async_copy(table_sh.at[sidx_v.at[j + b]], bufs[b],
                              sems[b]).wait()
        pltpu.sync_copy(bufs[b], acc_sh.at[didx_v.at[j + b]], add=True)

    for b in range(2):
      pltpu.make_async_copy(table_sh.at[sidx_v.at[NCH + b]], bufs[b],
                            sems[b]).wait()
    plsc.subcore_barrier()
    pltpu.sync_copy(acc_sh.at[pl.ds(s * RPT, RPT)],
                    out_hbm.at[c, pl.ds(s * RPT, RPT)])

  return _propagate


_propagate64 = _make_propagate(HID)
_propagate8 = _make_propagate(DP)


def _tc1_body(deg_ref, x_ref, w1_ref, hs_ref, dinv_ref):
  deg = deg_ref[0, :, 0:1] + deg_ref[1, :, 0:1] + 1.0
  dinv = lax.rsqrt(deg)
  h = jnp.dot(x_ref[...], w1_ref[...], preferred_element_type=jnp.float32)
  hs_ref[...] = h * dinv
  dinv_ref[...] = dinv


def _tc1(deg_p, x, W1):
  grid = (N // BR,)
  return pl.pallas_call(
      _tc1_body,
      grid=grid,
      in_specs=[
          pl.BlockSpec((NC, BR, DEG_W), lambda i: (0, i, 0)),
          pl.BlockSpec((BR, D_IN), lambda i: (i, 0)),
          pl.BlockSpec((D_IN, HID), lambda i: (0, 0)),
      ],
      out_specs=[
          pl.BlockSpec((BR, HID), lambda i: (i, 0)),
          pl.BlockSpec((BR, 1), lambda i: (i, 0)),
      ],
      out_shape=[
          jax.ShapeDtypeStruct((N, HID), jnp.float32),
          jax.ShapeDtypeStruct((N, 1), jnp.float32),
      ],
  )(deg_p, x, W1)


def _tc2_body(acc_ref, hs1_ref, dinv_ref, b1_ref, w2_ref, hs2_ref):
  acc = acc_ref[0] + acc_ref[1]
  dinv = dinv_ref[...]
  out1 = jnp.maximum((acc + hs1_ref[...]) * dinv + b1_ref[...], 0.0)
  h2 = jnp.dot(out1, w2_ref[...], preferred_element_type=jnp.float32)
  hs2_ref[...] = h2 * dinv


def _tc2(acc1, hs1, dinv, b1, W2):
  grid = (N // BR,)
  return pl.pallas_call(
      _tc2_body,
      grid=grid,
      in_specs=[
          pl.BlockSpec((NC, BR, HID), lambda i: (0, i, 0)),
          pl.BlockSpec((BR, HID), lambda i: (i, 0)),
          pl.BlockSpec((BR, 1), lambda i: (i, 0)),
          pl.BlockSpec((1, HID), lambda i: (0, 0)),
          pl.BlockSpec((HID, DP), lambda i: (0, 0)),
      ],
      out_specs=pl.BlockSpec((BR, DP), lambda i: (i, 0)),
      out_shape=jax.ShapeDtypeStruct((N, DP), jnp.float32),
  )(acc1, hs1, dinv, b1, W2)


def _tc3_body(acc_ref, hs2_ref, dinv_ref, b2_ref, out_ref):
  acc = acc_ref[0, :, 0:D_OUT] + acc_ref[1, :, 0:D_OUT]
  out_ref[...] = (acc + hs2_ref[:, 0:D_OUT]) * dinv_ref[...] + b2_ref[...]


def _tc3(acc2, hs2, dinv, b2):
  grid = (N // BR,)
  return pl.pallas_call(
      _tc3_body,
      grid=grid,
      in_specs=[
          pl.BlockSpec((NC, BR, DP), lambda i: (0, i, 0)),
          pl.BlockSpec((BR, DP), lambda i: (i, 0)),
          pl.BlockSpec((BR, 1), lambda i: (i, 0)),
          pl.BlockSpec((1, D_OUT), lambda i: (0, 0)),
      ],
      out_specs=pl.BlockSpec((BR, D_OUT), lambda i: (i, 0)),
      out_shape=jax.ShapeDtypeStruct((N, D_OUT), jnp.float32),
  )(acc2, hs2, dinv, b2)


def kernel(x, edge_index, W1, b1, W2, b2):
  src = edge_index[0].astype(jnp.int32).reshape(NW, E // NW)
  dst = edge_index[1].astype(jnp.int32).reshape(NW, E // NW)
  pad = EPW - E // NW
  src = jnp.pad(src, ((0, 0), (0, pad))).reshape(NW * NCH, CHUNK)
  dst = jnp.pad(dst, ((0, 0), (0, pad)), constant_values=N).reshape(
      NW * NCH, CHUNK)

  ones_b = jnp.ones((CHUNK, DEG_W), jnp.float32)
  zeros_deg = jnp.zeros((N_PAD, DEG_W), jnp.float32)
  zeros_h = jnp.zeros((N_PAD, HID), jnp.float32)
  zeros_o = jnp.zeros((N_PAD, DP), jnp.float32)

  deg_p = _deg_kernel(dst, ones_b, zeros_deg)
  hs1, dinv = _tc1(deg_p, x, W1)
  acc1 = _propagate64(hs1, src, dst, zeros_h)
  W2p = jnp.pad(W2, ((0, 0), (0, DP - D_OUT)))
  hs2 = _tc2(acc1, hs1, dinv, b1.reshape(1, HID), W2p)
  acc2 = _propagate8(hs2, src, dst, zeros_o)
  return _tc3(acc2, hs2, dinv, b2.reshape(1, D_OUT))


# speedup vs baseline: 1.0010x; 1.0010x over previous
import functools

import jax
import jax.numpy as jnp
from jax import lax
from jax.experimental import pallas as pl
from jax.experimental.pallas import tpu as pltpu
from jax.experimental.pallas import tpu_sc as plsc

N = 10000
E = 320000
D_IN = 128
HID = 64
D_OUT = 2

NC = 2
NS = 16
NW = NC * NS
CHUNK = 128
EPW = 10368
NCH = EPW // CHUNK
N_PAD = 10112
RPT = N_PAD // NS
DEG_W = 8
DP = 8
BR = 1000


def _sc_mesh():
  return plsc.VectorSubcoreMesh(core_axis_name="c", subcore_axis_name="s")


@functools.partial(
    pl.kernel,
    out_type=jax.ShapeDtypeStruct((NC, N_PAD, DEG_W), jnp.float32),
    mesh=_sc_mesh(),
    compiler_params=pltpu.CompilerParams(use_tc_tiling_on_sc=False),
    scratch_types=[
        pltpu.VMEM((NCH, CHUNK), jnp.int32),
        pltpu.VMEM((CHUNK, DEG_W), jnp.float32),
        pltpu.VMEM_SHARED((N_PAD, DEG_W), jnp.float32),
    ],
)
def _deg_kernel(dst_hbm, ones_hbm, zeros_hbm, out_hbm, didx_v, ones_v, acc_sh):
  c = lax.axis_index("c")
  s = lax.axis_index("s")
  wid = s * NC + c
  pltpu.sync_copy(dst_hbm.at[pl.ds(wid * NCH, NCH)], didx_v)
  pltpu.sync_copy(ones_hbm, ones_v)
  pltpu.sync_copy(zeros_hbm.at[pl.ds(s * RPT, RPT)],
                  acc_sh.at[pl.ds(s * RPT, RPT)])
  plsc.subcore_barrier()

  @pl.loop(0, NCH)
  def _(j):
    pltpu.sync_copy(ones_v, acc_sh.at[didx_v.at[j]], add=True)

  plsc.subcore_barrier()
  pltpu.sync_copy(acc_sh.at[pl.ds(s * RPT, RPT)],
                  out_hbm.at[c, pl.ds(s * RPT, RPT)])


def _make_propagate(D):
  @functools.partial(
      pl.kernel,
      out_type=jax.ShapeDtypeStruct((NC, N_PAD, D), jnp.float32),
      mesh=_sc_mesh(),
      compiler_params=pltpu.CompilerParams(use_tc_tiling_on_sc=False),
      scratch_types=[
          pltpu.VMEM((NCH + 2, CHUNK), jnp.int32),
          pltpu.VMEM((NCH, CHUNK), jnp.int32),
          pltpu.VMEM((CHUNK, D), jnp.float32),
          pltpu.VMEM((CHUNK, D), jnp.float32),
          pltpu.VMEM((CHUNK, D), jnp.float32),
          pltpu.VMEM_SHARED((N, D), jnp.float32),
          pltpu.VMEM_SHARED((N_PAD, D), jnp.float32),
          pltpu.SemaphoreType.DMA,
          pltpu.SemaphoreType.DMA,
          pltpu.SemaphoreType.DMA,
          pltpu.SemaphoreType.DMA,
          pltpu.SemaphoreType.DMA,
          pltpu.SemaphoreType.DMA,
      ],
  )
  def _propagate(table_hbm, src_hbm, dst_hbm, zeros_hbm, out_hbm,
                 sidx_v, didx_v, rows_0, rows_1, rows_2,
                 table_sh, acc_sh, sem_0, sem_1, sem_2,
                 ssem_0, ssem_1, ssem_2):
    bufs = (rows_0, rows_1, rows_2)
    sems = (sem_0, sem_1, sem_2)
    ssems = (ssem_0, ssem_1, ssem_2)
    c = lax.axis_index("c")
    s = lax.axis_index("s")
    wid = s * NC + c
    pltpu.sync_copy(src_hbm.at[pl.ds(wid * NCH, NCH)],
                    sidx_v.at[pl.ds(0, NCH)])
    pltpu.sync_copy(dst_hbm.at[pl.ds(wid * NCH, NCH)], didx_v)
    for t in range(2):
      for k in range(CHUNK // 16):
        sidx_v[NCH + t, pl.ds(k * 16, 16)] = jnp.zeros((16,), jnp.int32)
    pltpu.sync_copy(table_hbm.at[pl.ds(s * (N // NS), N // NS)],
                    table_sh.at[pl.ds(s * (N // NS), N // NS)])
    pltpu.sync_copy(zeros_hbm.at[pl.ds(s * RPT, RPT)],
                    acc_sh.at[pl.ds(s * RPT, RPT)])
    plsc.subcore_barrier()

    for b in range(2):
      pltpu.async_copy(table_sh.at[sidx_v.at[b]], bufs[b], sems[b])

    @pl.loop(0, NCH, step=3)
    def _(j):
      for b in range(3):
        jj = j + b
        nb = (b + 2) % 3
        @pl.when(jj >= 1)
        def _():
          pltpu.make_async_copy(bufs[nb], acc_sh.at[didx_v.at[jj - 1]],
                                ssems[nb]).wait()

        pltpu.async_copy(table_sh.at[sidx_v.at[jj + 2]], bufs[nb],
                         sems[nb])
        pltpu.make_async_copy(table_sh.at[sidx_v.at[jj]], bufs[b],
                              sems[b]).wait()
        pltpu.async_copy(bufs[b], acc_sh.at[didx_v.at[jj]], ssems[b],
                         add=True)

    for b in range(2):
      pltpu.make_async_copy(table_sh.at[sidx_v.at[NCH + b]], bufs[b],
                            sems[b]).wait()
    _sb = (NCH - 1) % 3
    pltpu.make_async_copy(bufs[_sb], acc_sh.at[didx_v.at[NCH - 1]],
                          ssems[_sb]).wait()
    plsc.subcore_barrier()
    pltpu.sync_copy(acc_sh.at[pl.ds(s * RPT, RPT)],
                    out_hbm.at[c, pl.ds(s * RPT, RPT)])

  return _propagate


_propagate64 = _make_propagate(HID)
_propagate8 = _make_propagate(DP)


def _tc1_body(deg_ref, x_ref, w1_ref, hs_ref, dinv_ref):
  deg = deg_ref[0, :, 0:1] + deg_ref[1, :, 0:1] + 1.0
  dinv = lax.rsqrt(deg)
  h = jnp.dot(x_ref[...], w1_ref[...], preferred_element_type=jnp.float32)
  hs_ref[...] = h * dinv
  dinv_ref[...] = dinv


def _tc1(deg_p, x, W1):
  grid = (N // BR,)
  return pl.pallas_call(
      _tc1_body,
      grid=grid,
      in_specs=[
          pl.BlockSpec((NC, BR, DEG_W), lambda i: (0, i, 0)),
          pl.BlockSpec((BR, D_IN), lambda i: (i, 0)),
          pl.BlockSpec((D_IN, HID), lambda i: (0, 0)),
      ],
      out_specs=[
          pl.BlockSpec((BR, HID), lambda i: (i, 0)),
          pl.BlockSpec((BR, 1), lambda i: (i, 0)),
      ],
      out_shape=[
          jax.ShapeDtypeStruct((N, HID), jnp.float32),
          jax.ShapeDtypeStruct((N, 1), jnp.float32),
      ],
  )(deg_p, x, W1)


def _tc2_body(acc_ref, hs1_ref, dinv_ref, b1_ref, w2_ref, hs2_ref):
  acc = acc_ref[0] + acc_ref[1]
  dinv = dinv_ref[...]
  out1 = jnp.maximum((acc + hs1_ref[...]) * dinv + b1_ref[...], 0.0)
  h2 = jnp.dot(out1, w2_ref[...], preferred_element_type=jnp.float32)
  hs2_ref[...] = h2 * dinv


def _tc2(acc1, hs1, dinv, b1, W2):
  grid = (N // BR,)
  return pl.pallas_call(
      _tc2_body,
      grid=grid,
      in_specs=[
          pl.BlockSpec((NC, BR, HID), lambda i: (0, i, 0)),
          pl.BlockSpec((BR, HID), lambda i: (i, 0)),
          pl.BlockSpec((BR, 1), lambda i: (i, 0)),
          pl.BlockSpec((1, HID), lambda i: (0, 0)),
          pl.BlockSpec((HID, DP), lambda i: (0, 0)),
      ],
      out_specs=pl.BlockSpec((BR, DP), lambda i: (i, 0)),
      out_shape=jax.ShapeDtypeStruct((N, DP), jnp.float32),
  )(acc1, hs1, dinv, b1, W2)


def _tc3_body(acc_ref, hs2_ref, dinv_ref, b2_ref, out_ref):
  acc = acc_ref[0, :, 0:D_OUT] + acc_ref[1, :, 0:D_OUT]
  out_ref[...] = (acc + hs2_ref[:, 0:D_OUT]) * dinv_ref[...] + b2_ref[...]


def _tc3(acc2, hs2, dinv, b2):
  grid = (N // BR,)
  return pl.pallas_call(
      _tc3_body,
      grid=grid,
      in_specs=[
          pl.BlockSpec((NC, BR, DP), lambda i: (0, i, 0)),
          pl.BlockSpec((BR, DP), lambda i: (i, 0)),
          pl.BlockSpec((BR, 1), lambda i: (i, 0)),
          pl.BlockSpec((1, D_OUT), lambda i: (0, 0)),
      ],
      out_specs=pl.BlockSpec((BR, D_OUT), lambda i: (i, 0)),
      out_shape=jax.ShapeDtypeStruct((N, D_OUT), jnp.float32),
  )(acc2, hs2, dinv, b2)


def kernel(x, edge_index, W1, b1, W2, b2):
  src = edge_index[0].astype(jnp.int32).reshape(NW, E // NW)
  dst = edge_index[1].astype(jnp.int32).reshape(NW, E // NW)
  pad = EPW - E // NW
  src = jnp.pad(src, ((0, 0), (0, pad))).reshape(NW * NCH, CHUNK)
  dst = jnp.pad(dst, ((0, 0), (0, pad)), constant_values=N).reshape(
      NW * NCH, CHUNK)

  ones_b = jnp.ones((CHUNK, DEG_W), jnp.float32)
  zeros_deg = jnp.zeros((N_PAD, DEG_W), jnp.float32)
  zeros_h = jnp.zeros((N_PAD, HID), jnp.float32)
  zeros_o = jnp.zeros((N_PAD, DP), jnp.float32)

  deg_p = _deg_kernel(dst, ones_b, zeros_deg)
  hs1, dinv = _tc1(deg_p, x, W1)
  acc1 = _propagate64(hs1, src, dst, zeros_h)
  W2p = jnp.pad(W2, ((0, 0), (0, DP - D_OUT)))
  hs2 = _tc2(acc1, hs1, dinv, b1.reshape(1, HID), W2p)
  acc2 = _propagate8(hs2, src, dst, zeros_o)
  return _tc3(acc2, hs2, dinv, b2.reshape(1, D_OUT))


# speedup vs baseline: 1.0229x; 1.0218x over previous
import functools

import jax
import jax.numpy as jnp
from jax import lax
from jax.experimental import pallas as pl
from jax.experimental.pallas import tpu as pltpu
from jax.experimental.pallas import tpu_sc as plsc

N = 10000
E = 320000
D_IN = 128
HID = 64
D_OUT = 2

NC = 2
NS = 16
NW = NC * NS
CHUNK = 128
EPW = 10240
NCH = EPW // CHUNK
N_PAD = 10112
RPT = N_PAD // NS
DEG_W = 8
DP = 8
BR = 1000


def _sc_mesh():
  return plsc.VectorSubcoreMesh(core_axis_name="c", subcore_axis_name="s")


@functools.partial(
    pl.kernel,
    out_type=jax.ShapeDtypeStruct((NC, N_PAD, DEG_W), jnp.float32),
    mesh=_sc_mesh(),
    compiler_params=pltpu.CompilerParams(use_tc_tiling_on_sc=False),
    scratch_types=[
        pltpu.VMEM((NCH, CHUNK), jnp.int32),
        pltpu.VMEM((CHUNK, DEG_W), jnp.float32),
        pltpu.VMEM_SHARED((N_PAD, DEG_W), jnp.float32),
    ],
)
def _deg_kernel(dst_hbm, ones_hbm, zeros_hbm, out_hbm, didx_v, ones_v, acc_sh):
  c = lax.axis_index("c")
  s = lax.axis_index("s")
  wid = s * NC + c
  pltpu.sync_copy(dst_hbm.at[pl.ds(wid * NCH, NCH)], didx_v)
  pltpu.sync_copy(ones_hbm, ones_v)
  pltpu.sync_copy(zeros_hbm.at[pl.ds(s * RPT, RPT)],
                  acc_sh.at[pl.ds(s * RPT, RPT)])
  plsc.subcore_barrier()

  @pl.loop(0, NCH)
  def _(j):
    pltpu.sync_copy(ones_v, acc_sh.at[didx_v.at[j]], add=True)

  plsc.subcore_barrier()
  pltpu.sync_copy(acc_sh.at[pl.ds(s * RPT, RPT)],
                  out_hbm.at[c, pl.ds(s * RPT, RPT)])


def _make_propagate(D):
  @functools.partial(
      pl.kernel,
      out_type=jax.ShapeDtypeStruct((NC, N_PAD, D), jnp.float32),
      mesh=_sc_mesh(),
      compiler_params=pltpu.CompilerParams(use_tc_tiling_on_sc=False),
      scratch_types=[
          pltpu.VMEM((NCH + 1, CHUNK), jnp.int32),
          pltpu.VMEM((NCH, CHUNK), jnp.int32),
          pltpu.VMEM((CHUNK, D), jnp.float32),
          pltpu.VMEM((CHUNK, D), jnp.float32),
          pltpu.VMEM_SHARED((N, D), jnp.float32),
          pltpu.VMEM_SHARED((N_PAD, D), jnp.float32),
          pltpu.SemaphoreType.DMA,
          pltpu.SemaphoreType.DMA,
      ],
  )
  def _propagate(table_hbm, src_hbm, dst_hbm, zeros_hbm, out_hbm,
                 sidx_v, didx_v, rows_a, rows_b,
                 table_sh, acc_sh, sem_a, sem_b):
    c = lax.axis_index("c")
    s = lax.axis_index("s")
    wid = s * NC + c
    pltpu.sync_copy(src_hbm.at[pl.ds(wid * NCH, NCH)],
                    sidx_v.at[pl.ds(0, NCH)])
    pltpu.sync_copy(dst_hbm.at[pl.ds(wid * NCH, NCH)], didx_v)
    for k in range(CHUNK // 16):
      sidx_v[NCH, pl.ds(k * 16, 16)] = jnp.zeros((16,), jnp.int32)
    pltpu.sync_copy(table_hbm.at[pl.ds(s * (N // NS), N // NS)],
                    table_sh.at[pl.ds(s * (N // NS), N // NS)])
    pltpu.sync_copy(zeros_hbm.at[pl.ds(s * RPT, RPT)],
                    acc_sh.at[pl.ds(s * RPT, RPT)])
    plsc.subcore_barrier()

    pltpu.async_copy(table_sh.at[sidx_v.at[0]], rows_a, sem_a)

    @pl.loop(0, NCH, step=2)
    def _(j):
      pltpu.async_copy(table_sh.at[sidx_v.at[j + 1]], rows_b, sem_b)
      pltpu.make_async_copy(table_sh.at[sidx_v.at[j]], rows_a, sem_a).wait()
      pltpu.sync_copy(rows_a, acc_sh.at[didx_v.at[j]], add=True)
      pltpu.async_copy(table_sh.at[sidx_v.at[j + 2]], rows_a, sem_a)
      pltpu.make_async_copy(table_sh.at[sidx_v.at[j + 1]], rows_b,
                            sem_b).wait()
      pltpu.sync_copy(rows_b, acc_sh.at[didx_v.at[j + 1]], add=True)

    pltpu.make_async_copy(table_sh.at[sidx_v.at[NCH]], rows_a, sem_a).wait()
    plsc.subcore_barrier()
    pltpu.sync_copy(acc_sh.at[pl.ds(s * RPT, RPT)],
                    out_hbm.at[c, pl.ds(s * RPT, RPT)])

  return _propagate


_propagate64 = _make_propagate(HID)
_propagate8 = _make_propagate(DP)


def _tc1_body(deg_ref, x_ref, w1_ref, hs_ref, dinv_ref):
  deg = deg_ref[0, :, 0:1] + deg_ref[1, :, 0:1] + 1.0
  dinv = lax.rsqrt(deg)
  h = jnp.dot(x_ref[...], w1_ref[...], preferred_element_type=jnp.float32)
  hs_ref[...] = h * dinv
  dinv_ref[...] = dinv


def _tc1(deg_p, x, W1):
  grid = (N // BR,)
  return pl.pallas_call(
      _tc1_body,
      grid=grid,
      in_specs=[
          pl.BlockSpec((NC, BR, DEG_W), lambda i: (0, i, 0)),
          pl.BlockSpec((BR, D_IN), lambda i: (i, 0)),
          pl.BlockSpec((D_IN, HID), lambda i: (0, 0)),
      ],
      out_specs=[
          pl.BlockSpec((BR, HID), lambda i: (i, 0)),
          pl.BlockSpec((BR, 1), lambda i: (i, 0)),
      ],
      out_shape=[
          jax.ShapeDtypeStruct((N, HID), jnp.float32),
          jax.ShapeDtypeStruct((N, 1), jnp.float32),
      ],
  )(deg_p, x, W1)


def _tc2_body(acc_ref, hs1_ref, dinv_ref, b1_ref, w2_ref, hs2_ref):
  acc = acc_ref[0] + acc_ref[1]
  dinv = dinv_ref[...]
  out1 = jnp.maximum((acc + hs1_ref[...]) * dinv + b1_ref[...], 0.0)
  h2 = jnp.dot(out1, w2_ref[...], preferred_element_type=jnp.float32)
  hs2_ref[...] = h2 * dinv


def _tc2(acc1, hs1, dinv, b1, W2):
  grid = (N // BR,)
  return pl.pallas_call(
      _tc2_body,
      grid=grid,
      in_specs=[
          pl.BlockSpec((NC, BR, HID), lambda i: (0, i, 0)),
          pl.BlockSpec((BR, HID), lambda i: (i, 0)),
          pl.BlockSpec((BR, 1), lambda i: (i, 0)),
          pl.BlockSpec((1, HID), lambda i: (0, 0)),
          pl.BlockSpec((HID, DP), lambda i: (0, 0)),
      ],
      out_specs=pl.BlockSpec((BR, DP), lambda i: (i, 0)),
      out_shape=jax.ShapeDtypeStruct((N, DP), jnp.float32),
  )(acc1, hs1, dinv, b1, W2)


def _tc3_body(acc_ref, hs2_ref, dinv_ref, b2_ref, out_ref):
  acc = acc_ref[0, :, 0:D_OUT] + acc_ref[1, :, 0:D_OUT]
  out_ref[...] = (acc + hs2_ref[:, 0:D_OUT]) * dinv_ref[...] + b2_ref[...]


def _tc3(acc2, hs2, dinv, b2):
  grid = (N // BR,)
  return pl.pallas_call(
      _tc3_body,
      grid=grid,
      in_specs=[
          pl.BlockSpec((NC, BR, DP), lambda i: (0, i, 0)),
          pl.BlockSpec((BR, DP), lambda i: (i, 0)),
          pl.BlockSpec((BR, 1), lambda i: (i, 0)),
          pl.BlockSpec((1, D_OUT), lambda i: (0, 0)),
      ],
      out_specs=pl.BlockSpec((BR, D_OUT), lambda i: (i, 0)),
      out_shape=jax.ShapeDtypeStruct((N, D_OUT), jnp.float32),
  )(acc2, hs2, dinv, b2)


def kernel(x, edge_index, W1, b1, W2, b2):
  src = edge_index[0].astype(jnp.int32).reshape(NW, E // NW)
  dst = edge_index[1].astype(jnp.int32).reshape(NW, E // NW)
  pad = EPW - E // NW
  src = jnp.pad(src, ((0, 0), (0, pad))).reshape(NW * NCH, CHUNK)
  dst = jnp.pad(dst, ((0, 0), (0, pad)), constant_values=N).reshape(
      NW * NCH, CHUNK)

  ones_b = jnp.ones((CHUNK, DEG_W), jnp.float32)
  zeros_deg = jnp.zeros((N_PAD, DEG_W), jnp.float32)
  zeros_h = jnp.zeros((N_PAD, HID), jnp.float32)
  zeros_o = jnp.zeros((N_PAD, DP), jnp.float32)

  deg_p = _deg_kernel(dst, ones_b, zeros_deg)
  hs1, dinv = _tc1(deg_p, x, W1)
  acc1 = _propagate64(hs1, src, dst, zeros_h)
  W2p = jnp.pad(W2, ((0, 0), (0, DP - D_OUT)))
  hs2 = _tc2(acc1, hs1, dinv, b1.reshape(1, HID), W2p)
  acc2 = _propagate8(hs2, src, dst, zeros_o)
  return _tc3(acc2, hs2, dinv, b2.reshape(1, D_OUT))


# speedup vs baseline: 1.0244x; 1.0015x over previous
import functools

import jax
import jax.numpy as jnp
from jax import lax
from jax.experimental import pallas as pl
from jax.experimental.pallas import tpu as pltpu
from jax.experimental.pallas import tpu_sc as plsc

N = 10000
E = 320000
D_IN = 128
HID = 64
D_OUT = 2

NC = 2
NS = 16
NW = NC * NS
CHUNK = 128
EPW = 10240
NCH = EPW // CHUNK
N_PAD = 10112
RPT = N_PAD // NS
DEG_W = 8
DP = 8
BR = 1000


def _sc_mesh():
  return plsc.VectorSubcoreMesh(core_axis_name="c", subcore_axis_name="s",
                                num_cores=NC, num_subcores=NS)


@functools.partial(
    pl.kernel,
    out_type=jax.ShapeDtypeStruct((NC, N_PAD, DEG_W), jnp.float32),
    mesh=_sc_mesh(),
    compiler_params=pltpu.CompilerParams(use_tc_tiling_on_sc=False),
    scratch_types=[
        pltpu.VMEM((NCH, CHUNK), jnp.int32),
        pltpu.VMEM((CHUNK, DEG_W), jnp.float32),
        pltpu.VMEM_SHARED((N_PAD, DEG_W), jnp.float32),
    ],
)
def _deg_kernel(dst_hbm, ones_hbm, zeros_hbm, out_hbm, didx_v, ones_v, acc_sh):
  c = lax.axis_index("c")
  s = lax.axis_index("s")
  wid = s * NC + c
  pltpu.sync_copy(dst_hbm.at[pl.ds(wid * NCH, NCH)], didx_v)
  pltpu.sync_copy(ones_hbm, ones_v)
  pltpu.sync_copy(zeros_hbm.at[pl.ds(s * RPT, RPT)],
                  acc_sh.at[pl.ds(s * RPT, RPT)])
  plsc.subcore_barrier()

  @pl.loop(0, NCH)
  def _(j):
    pltpu.sync_copy(ones_v, acc_sh.at[didx_v.at[j]], add=True)

  plsc.subcore_barrier()
  pltpu.sync_copy(acc_sh.at[pl.ds(s * RPT, RPT)],
                  out_hbm.at[c, pl.ds(s * RPT, RPT)])


def _make_propagate(D):
  @functools.partial(
      pl.kernel,
      out_type=jax.ShapeDtypeStruct((NC, N_PAD, D), jnp.float32),
      mesh=_sc_mesh(),
      compiler_params=pltpu.CompilerParams(use_tc_tiling_on_sc=False),
      scratch_types=[
          pltpu.VMEM((NCH + 1, CHUNK), jnp.int32),
          pltpu.VMEM((NCH, CHUNK), jnp.int32),
          pltpu.VMEM((CHUNK, D), jnp.float32),
          pltpu.VMEM((CHUNK, D), jnp.float32),
          pltpu.VMEM_SHARED((N, D), jnp.float32),
          pltpu.VMEM_SHARED((N_PAD, D), jnp.float32),
          pltpu.SemaphoreType.DMA,
          pltpu.SemaphoreType.DMA,
      ],
  )
  def _propagate(table_hbm, src_hbm, dst_hbm, zeros_hbm, out_hbm,
                 sidx_v, didx_v, rows_a, rows_b,
                 table_sh, acc_sh, sem_a, sem_b):
    c = lax.axis_index("c")
    s = lax.axis_index("s")
    wid = s * NC + c
    pltpu.sync_copy(src_hbm.at[pl.ds(wid * NCH, NCH)],
                    sidx_v.at[pl.ds(0, NCH)])
    pltpu.sync_copy(dst_hbm.at[pl.ds(wid * NCH, NCH)], didx_v)
    for k in range(CHUNK // 16):
      sidx_v[NCH, pl.ds(k * 16, 16)] = jnp.zeros((16,), jnp.int32)
    pltpu.sync_copy(table_hbm.at[pl.ds(s * (N // NS), N // NS)],
                    table_sh.at[pl.ds(s * (N // NS), N // NS)])
    pltpu.sync_copy(zeros_hbm.at[pl.ds(s * RPT, RPT)],
                    acc_sh.at[pl.ds(s * RPT, RPT)])
    plsc.subcore_barrier()

    pltpu.async_copy(table_sh.at[sidx_v.at[0]], rows_a, sem_a)

    @pl.loop(0, NCH, step=2)
    def _(j):
      pltpu.async_copy(table_sh.at[sidx_v.at[j + 1]], rows_b, sem_b)
      pltpu.make_async_copy(table_sh.at[sidx_v.at[j]], rows_a, sem_a).wait()
      pltpu.sync_copy(rows_a, acc_sh.at[didx_v.at[j]], add=True)
      pltpu.async_copy(table_sh.at[sidx_v.at[j + 2]], rows_a, sem_a)
      pltpu.make_async_copy(table_sh.at[sidx_v.at[j + 1]], rows_b,
                            sem_b).wait()
      pltpu.sync_copy(rows_b, acc_sh.at[didx_v.at[j + 1]], add=True)

    pltpu.make_async_copy(table_sh.at[sidx_v.at[NCH]], rows_a, sem_a).wait()
    plsc.subcore_barrier()
    pltpu.sync_copy(acc_sh.at[pl.ds(s * RPT, RPT)],
                    out_hbm.at[c, pl.ds(s * RPT, RPT)])

  return _propagate


_propagate64 = _make_propagate(HID)
_propagate8 = _make_propagate(DP)


def _tc1_body(deg_ref, x_ref, w1_ref, hs_ref, dinv_ref):
  deg = deg_ref[0, :, 0:1] + deg_ref[1, :, 0:1] + 1.0
  dinv = lax.rsqrt(deg)
  h = jnp.dot(x_ref[...], w1_ref[...], preferred_element_type=jnp.float32)
  hs_ref[...] = h * dinv
  dinv_ref[...] = dinv


def _tc1(deg_p, x, W1):
  grid = (N // BR,)
  return pl.pallas_call(
      _tc1_body,
      grid=grid,
      in_specs=[
          pl.BlockSpec((NC, BR, DEG_W), lambda i: (0, i, 0)),
          pl.BlockSpec((BR, D_IN), lambda i: (i, 0)),
          pl.BlockSpec((D_IN, HID), lambda i: (0, 0)),
      ],
      out_specs=[
          pl.BlockSpec((BR, HID), lambda i: (i, 0)),
          pl.BlockSpec((BR, 1), lambda i: (i, 0)),
      ],
      out_shape=[
          jax.ShapeDtypeStruct((N, HID), jnp.float32),
          jax.ShapeDtypeStruct((N, 1), jnp.float32),
      ],
  )(deg_p, x, W1)


def _tc2_body(acc_ref, hs1_ref, dinv_ref, b1_ref, w2_ref, hs2_ref):
  acc = acc_ref[0] + acc_ref[1]
  dinv = dinv_ref[...]
  out1 = jnp.maximum((acc + hs1_ref[...]) * dinv + b1_ref[...], 0.0)
  h2 = jnp.dot(out1, w2_ref[...], preferred_element_type=jnp.float32)
  hs2_ref[...] = h2 * dinv


def _tc2(acc1, hs1, dinv, b1, W2):
  grid = (N // BR,)
  return pl.pallas_call(
      _tc2_body,
      grid=grid,
      in_specs=[
          pl.BlockSpec((NC, BR, HID), lambda i: (0, i, 0)),
          pl.BlockSpec((BR, HID), lambda i: (i, 0)),
          pl.BlockSpec((BR, 1), lambda i: (i, 0)),
          pl.BlockSpec((1, HID), lambda i: (0, 0)),
          pl.BlockSpec((HID, DP), lambda i: (0, 0)),
      ],
      out_specs=pl.BlockSpec((BR, DP), lambda i: (i, 0)),
      out_shape=jax.ShapeDtypeStruct((N, DP), jnp.float32),
  )(acc1, hs1, dinv, b1, W2)


def _tc3_body(acc_ref, hs2_ref, dinv_ref, b2_ref, out_ref):
  acc = acc_ref[0, :, 0:D_OUT] + acc_ref[1, :, 0:D_OUT]
  out_ref[...] = (acc + hs2_ref[:, 0:D_OUT]) * dinv_ref[...] + b2_ref[...]


def _tc3(acc2, hs2, dinv, b2):
  grid = (N // BR,)
  return pl.pallas_call(
      _tc3_body,
      grid=grid,
      in_specs=[
          pl.BlockSpec((NC, BR, DP), lambda i: (0, i, 0)),
          pl.BlockSpec((BR, DP), lambda i: (i, 0)),
          pl.BlockSpec((BR, 1), lambda i: (i, 0)),
          pl.BlockSpec((1, D_OUT), lambda i: (0, 0)),
      ],
      out_specs=pl.BlockSpec((BR, D_OUT), lambda i: (i, 0)),
      out_shape=jax.ShapeDtypeStruct((N, D_OUT), jnp.float32),
  )(acc2, hs2, dinv, b2)


def kernel(x, edge_index, W1, b1, W2, b2):
  src = edge_index[0].astype(jnp.int32).reshape(NW, E // NW)
  dst = edge_index[1].astype(jnp.int32).reshape(NW, E // NW)
  pad = EPW - E // NW
  src = jnp.pad(src, ((0, 0), (0, pad))).reshape(NW * NCH, CHUNK)
  dst = jnp.pad(dst, ((0, 0), (0, pad)), constant_values=N).reshape(
      NW * NCH, CHUNK)

  ones_b = jnp.ones((CHUNK, DEG_W), jnp.float32)
  zeros_deg = jnp.zeros((N_PAD, DEG_W), jnp.float32)
  zeros_h = jnp.zeros((N_PAD, HID), jnp.float32)
  zeros_o = jnp.zeros((N_PAD, DP), jnp.float32)

  deg_p = _deg_kernel(dst, ones_b, zeros_deg)
  hs1, dinv = _tc1(deg_p, x, W1)
  acc1 = _propagate64(hs1, src, dst, zeros_h)
  W2p = jnp.pad(W2, ((0, 0), (0, DP - D_OUT)))
  hs2 = _tc2(acc1, hs1, dinv, b1.reshape(1, HID), W2p)
  acc2 = _propagate8(hs2, src, dst, zeros_o)
  return _tc3(acc2, hs2, dinv, b2.reshape(1, D_OUT))
